# Initial kernel scaffold; baseline (speedup 1.0000x reference)
import jax
import jax.numpy as jnp
from jax.experimental import pallas as pl


def kernel(x, edge_index, W1l, W1r, b1, W2l, W2r, b2, Wc1, bc1, Wc2, bc2):
    raise NotImplementedError("write your pallas kernel here")


# speedup vs baseline: 6.2984x; 6.2984x over previous
import functools

import jax
import jax.numpy as jnp
from jax import lax
from jax.experimental import pallas as pl
from jax.experimental.pallas import tpu as pltpu
from jax.experimental.pallas import tpu_sc as plsc

F32 = jnp.float32
I32 = jnp.int32

N = 10000
NA = 10240
E = 320000
C = 128
NCORE = 2
NSUB = 16
NW = NCORE * NSUB
K = -(-E // (NW * C))
EP = NW * C * K
JUNK = NA - 8
RPS = NA // NSUB
BM = 1024


def _sc_body(with_deg, d, *refs):
    if with_deg:
        (p_hbm, src_hbm, dst_hbm, zrow_hbm, zvec_hbm,
         agg_hbm, deg_hbm,
         accum_s, src_v, dst_v, rows_v, sem, deg_s, ones_v) = refs
    else:
        (p_hbm, src_hbm, dst_hbm, zrow_hbm,
         agg_hbm,
         accum_s, src_v, dst_v, rows_v, sem) = refs
    c = lax.axis_index("c")
    s = lax.axis_index("s")
    wid = c * NSUB + s

    zbase = s * RPS
    for j in range(RPS // 128):
        pltpu.sync_copy(zrow_hbm, accum_s.at[pl.ds(zbase + j * 128, 128)])
    if with_deg:
        pltpu.sync_copy(zvec_hbm, deg_s.at[pl.ds(zbase, RPS)])
        for j in range(C // 16):
            ones_v[pl.ds(j * 16, 16)] = jnp.full((16,), 1.0, F32)
    plsc.subcore_barrier()

    def chunk(k, carry):
        off = (wid * K + k) * C
        pltpu.sync_copy(src_hbm.at[pl.ds(off, C)], src_v)
        pltpu.sync_copy(dst_hbm.at[pl.ds(off, C)], dst_v)
        pltpu.async_copy(p_hbm.at[src_v], rows_v, sem).wait()
        pltpu.sync_copy(rows_v, accum_s.at[dst_v], add=True)
        if with_deg:
            pltpu.sync_copy(ones_v, deg_s.at[dst_v], add=True)
        return carry

    lax.fori_loop(0, K, chunk, 0)
    plsc.subcore_barrier()

    wb = s * RPS
    pltpu.sync_copy(accum_s.at[pl.ds(wb, RPS)], agg_hbm.at[c, pl.ds(wb, RPS)])
    if with_deg:
        pltpu.sync_copy(deg_s.at[pl.ds(wb, RPS)], deg_hbm.at[c, pl.ds(wb, RPS)])


def _sc_segment_sum(p, src, dst, d, with_deg):
    mesh = plsc.VectorSubcoreMesh(core_axis_name="c", subcore_axis_name="s",
                                  num_cores=NCORE, num_subcores=NSUB)
    outs = [jax.ShapeDtypeStruct((NCORE, NA, d), F32)]
    scratch = [
        pltpu.VMEM_SHARED((NA, d), F32),
        pltpu.VMEM((C,), I32),
        pltpu.VMEM((C,), I32),
        pltpu.VMEM((C, d), F32),
        pltpu.SemaphoreType.DMA,
    ]
    operands = [p, src, dst, jnp.zeros((128, d), F32)]
    if with_deg:
        outs.append(jax.ShapeDtypeStruct((NCORE, NA), F32))
        scratch += [pltpu.VMEM_SHARED((NA,), F32),
                    pltpu.VMEM((C,), F32)]
        operands.append(jnp.zeros((RPS,), F32))
    fn = pl.kernel(functools.partial(_sc_body, with_deg, d),
                   out_type=tuple(outs), mesh=mesh, scratch_types=scratch,
                   compiler_params=pltpu.CompilerParams(
                       use_tc_tiling_on_sc=False))
    return fn(*operands)


def _tc_pre_body(x_ref, wl_ref, wr_ref, b_ref, p_ref, s_ref):
    xb = x_ref[...]
    p_ref[...] = jnp.dot(xb, wl_ref[...], preferred_element_type=F32)
    s_ref[...] = jnp.dot(xb, wr_ref[...], preferred_element_type=F32) + b_ref[...]


def _tc_mid_body(a0, a1, d0, d1, s1, wl, wr, b, p_ref, s_ref):
    inv = 1.0 / jnp.maximum(d0[...] + d1[...], 1.0)
    h = jnp.maximum((a0[...] + a1[...]) * inv + s1[...], 0.0)
    p_ref[...] = jnp.dot(h, wl[...], preferred_element_type=F32)
    s_ref[...] = jnp.dot(h, wr[...], preferred_element_type=F32) + b[...]


def _tc_post_body(a0, a1, d0, d1, s2, w1, b1, w2, b2, o_ref):
    inv = 1.0 / jnp.maximum(d0[...] + d1[...], 1.0)
    h = jnp.maximum((a0[...] + a1[...]) * inv + s2[...], 0.0)
    h = jnp.maximum(jnp.dot(h, w1[...], preferred_element_type=F32) + b1[...], 0.0)
    o_ref[...] = jnp.dot(h, w2[...], preferred_element_type=F32) + b2[...]


def _row_spec(dcols):
    return pl.BlockSpec((BM, dcols), lambda i: (i, 0))


def _full_spec(r, c):
    return pl.BlockSpec((r, c), lambda i: (0, 0))


def _tc_pre(xp, wl, wr, b):
    return pl.pallas_call(
        _tc_pre_body,
        grid=(NA // BM,),
        in_specs=[_row_spec(128), _full_spec(128, 64), _full_spec(128, 64),
                  _full_spec(1, 64)],
        out_specs=[_row_spec(64), _row_spec(64)],
        out_shape=[jax.ShapeDtypeStruct((NA, 64), F32)] * 2,
    )(xp, wl, wr, b)


def _tc_mid(a0, a1, d0, d1, s1, wl, wr, b):
    return pl.pallas_call(
        _tc_mid_body,
        grid=(NA // BM,),
        in_specs=[_row_spec(64), _row_spec(64), _row_spec(1), _row_spec(1),
                  _row_spec(64), _full_spec(64, 32), _full_spec(64, 32),
                  _full_spec(1, 32)],
        out_specs=[_row_spec(32), _row_spec(32)],
        out_shape=[jax.ShapeDtypeStruct((NA, 32), F32)] * 2,
    )(a0, a1, d0, d1, s1, wl, wr, b)


def _tc_post(a0, a1, d0, d1, s2, w1, b1, w2, b2):
    return pl.pallas_call(
        _tc_post_body,
        grid=(NA // BM,),
        in_specs=[_row_spec(32), _row_spec(32), _row_spec(1), _row_spec(1),
                  _row_spec(32), _full_spec(32, 16), _full_spec(1, 16),
                  _full_spec(16, 2), _full_spec(1, 2)],
        out_specs=_row_spec(2),
        out_shape=jax.ShapeDtypeStruct((NA, 2), F32),
    )(a0, a1, d0, d1, s2, w1, b1, w2, b2)


def kernel(x, edge_index, W1l, W1r, b1, W2l, W2r, b2, Wc1, bc1, Wc2, bc2):
    x = x.astype(F32)
    ei = edge_index.astype(I32)
    src = jnp.concatenate([ei[0], jnp.zeros((EP - E,), I32)])
    dst = jnp.concatenate([ei[1], jnp.full((EP - E,), JUNK, I32)])
    xp = jnp.pad(x, ((0, NA - N), (0, 0)))

    p1, s1 = _tc_pre(xp, W1l.T, W1r.T, b1.reshape(1, -1))
    agg1, deg = _sc_segment_sum(p1, src, dst, 64, with_deg=True)
    d0 = deg[0].reshape(NA, 1)
    d1 = deg[1].reshape(NA, 1)
    p2, s2 = _tc_mid(agg1[0], agg1[1], d0, d1, s1,
                     W2l.T, W2r.T, b2.reshape(1, -1))
    (agg2,) = _sc_segment_sum(p2, src, dst, 32, with_deg=False)
    out = _tc_post(agg2[0], agg2[1], d0, d1, s2,
                   Wc1.T, bc1.reshape(1, -1), Wc2.T, bc2.reshape(1, -1))
    return out[:N]


# speedup vs baseline: 6.6917x; 1.0624x over previous
import functools

import jax
import jax.numpy as jnp
from jax import lax
from jax.experimental import pallas as pl
from jax.experimental.pallas import tpu as pltpu
from jax.experimental.pallas import tpu_sc as plsc

F32 = jnp.float32
I32 = jnp.int32

N = 10000
NA = 10240
E = 320000
C = 128
NCORE = 2
NSUB = 16
NW = NCORE * NSUB
K = 80
EP = NW * C * K
JUNK = NA - 8
RPS = NA // NSUB
BM = 1024


def _sc_body(with_deg, d, *refs):
    if with_deg:
        (p_hbm, src_hbm, dst_hbm, zrow_hbm, zvec_hbm,
         agg_hbm, deg_hbm,
         accum_s, src_v, dst_v, rows0, rows1,
         gsem0, gsem1, ssem0, ssem1, deg_s, ones_v) = refs
    else:
        (p_hbm, src_hbm, dst_hbm, zrow_hbm,
         agg_hbm,
         accum_s, src_v, dst_v, rows0, rows1,
         gsem0, gsem1, ssem0, ssem1) = refs
    rows = (rows0, rows1)
    gsem = (gsem0, gsem1)
    ssem = (ssem0, ssem1)
    c = lax.axis_index("c")
    s = lax.axis_index("s")
    wid = c * NSUB + s

    zbase = s * RPS
    for j in range(RPS // 128):
        pltpu.sync_copy(zrow_hbm, accum_s.at[pl.ds(zbase + j * 128, 128)])
    if with_deg:
        pltpu.sync_copy(zvec_hbm, deg_s.at[pl.ds(zbase, RPS)])
        for j in range(C // 16):
            ones_v[pl.ds(j * 16, 16)] = jnp.full((16,), 1.0, F32)

    pltpu.sync_copy(src_hbm.at[pl.ds(wid * K, K)], src_v)
    pltpu.sync_copy(dst_hbm.at[pl.ds(wid * K, K)], dst_v)
    plsc.subcore_barrier()

    pltpu.async_copy(p_hbm.at[src_v.at[0]], rows0, gsem0)

    def outer(j2, carry):
        for b in range(2):
            j = j2 * 2 + b
            pltpu.make_async_copy(p_hbm.at[src_v.at[j]], rows[b],
                                  gsem[b]).wait()
            pltpu.async_copy(rows[b], accum_s.at[dst_v.at[j]], ssem[b],
                             add=True)
            if with_deg:
                pltpu.sync_copy(ones_v, deg_s.at[dst_v.at[j]], add=True)

            @pl.when(j + 1 < K)
            def _issue():
                @pl.when(j >= 1)
                def _drain():
                    pltpu.make_async_copy(rows[1 - b],
                                          accum_s.at[dst_v.at[j - 1]],
                                          ssem[1 - b]).wait()
                pltpu.async_copy(p_hbm.at[src_v.at[j + 1]], rows[1 - b],
                                 gsem[1 - b])
        return carry

    lax.fori_loop(0, K // 2, outer, 0)
    pltpu.make_async_copy(rows0, accum_s.at[dst_v.at[K - 2]], ssem0).wait()
    pltpu.make_async_copy(rows1, accum_s.at[dst_v.at[K - 1]], ssem1).wait()
    plsc.subcore_barrier()

    wb = s * RPS
    pltpu.sync_copy(accum_s.at[pl.ds(wb, RPS)], agg_hbm.at[c, pl.ds(wb, RPS)])
    if with_deg:
        pltpu.sync_copy(deg_s.at[pl.ds(wb, RPS)], deg_hbm.at[c, pl.ds(wb, RPS)])


def _sc_segment_sum(p, src, dst, d, with_deg):
    mesh = plsc.VectorSubcoreMesh(core_axis_name="c", subcore_axis_name="s",
                                  num_cores=NCORE, num_subcores=NSUB)
    outs = [jax.ShapeDtypeStruct((NCORE, NA, d), F32)]
    scratch = [
        pltpu.VMEM_SHARED((NA, d), F32),
        pltpu.VMEM((K, C), I32),
        pltpu.VMEM((K, C), I32),
        pltpu.VMEM((C, d), F32),
        pltpu.VMEM((C, d), F32),
        pltpu.SemaphoreType.DMA,
        pltpu.SemaphoreType.DMA,
        pltpu.SemaphoreType.DMA,
        pltpu.SemaphoreType.DMA,
    ]
    operands = [p, src, dst, jnp.zeros((128, d), F32)]
    if with_deg:
        outs.append(jax.ShapeDtypeStruct((NCORE, NA), F32))
        scratch += [pltpu.VMEM_SHARED((NA,), F32),
                    pltpu.VMEM((C,), F32)]
        operands.append(jnp.zeros((RPS,), F32))
    fn = pl.kernel(functools.partial(_sc_body, with_deg, d),
                   out_type=tuple(outs), mesh=mesh, scratch_types=scratch,
                   compiler_params=pltpu.CompilerParams(
                       use_tc_tiling_on_sc=False))
    return fn(*operands)


def _tc_pre_body(x_ref, wl_ref, wr_ref, b_ref, p_ref, s_ref):
    xb = x_ref[...]
    p_ref[...] = jnp.dot(xb, wl_ref[...], preferred_element_type=F32)
    s_ref[...] = jnp.dot(xb, wr_ref[...], preferred_element_type=F32) + b_ref[...]


def _tc_mid_body(a0, a1, d0, d1, s1, wl, wr, b, p_ref, s_ref):
    inv = 1.0 / jnp.maximum(d0[...] + d1[...], 1.0)
    h = jnp.maximum((a0[...] + a1[...]) * inv + s1[...], 0.0)
    p_ref[...] = jnp.dot(h, wl[...], preferred_element_type=F32)
    s_ref[...] = jnp.dot(h, wr[...], preferred_element_type=F32) + b[...]


def _tc_post_body(a0, a1, d0, d1, s2, w1, b1, w2, b2, o_ref):
    inv = 1.0 / jnp.maximum(d0[...] + d1[...], 1.0)
    h = jnp.maximum((a0[...] + a1[...]) * inv + s2[...], 0.0)
    h = jnp.maximum(jnp.dot(h, w1[...], preferred_element_type=F32) + b1[...], 0.0)
    o_ref[...] = jnp.dot(h, w2[...], preferred_element_type=F32) + b2[...]


def _row_spec(dcols):
    return pl.BlockSpec((BM, dcols), lambda i: (i, 0))


def _full_spec(r, c):
    return pl.BlockSpec((r, c), lambda i: (0, 0))


def _tc_pre(xp, wl, wr, b):
    return pl.pallas_call(
        _tc_pre_body,
        grid=(NA // BM,),
        in_specs=[_row_spec(128), _full_spec(128, 64), _full_spec(128, 64),
                  _full_spec(1, 64)],
        out_specs=[_row_spec(64), _row_spec(64)],
        out_shape=[jax.ShapeDtypeStruct((NA, 64), F32)] * 2,
    )(xp, wl, wr, b)


def _tc_mid(a0, a1, d0, d1, s1, wl, wr, b):
    return pl.pallas_call(
        _tc_mid_body,
        grid=(NA // BM,),
        in_specs=[_row_spec(64), _row_spec(64), _row_spec(1), _row_spec(1),
                  _row_spec(64), _full_spec(64, 32), _full_spec(64, 32),
                  _full_spec(1, 32)],
        out_specs=[_row_spec(32), _row_spec(32)],
        out_shape=[jax.ShapeDtypeStruct((NA, 32), F32)] * 2,
    )(a0, a1, d0, d1, s1, wl, wr, b)


def _tc_post(a0, a1, d0, d1, s2, w1, b1, w2, b2):
    return pl.pallas_call(
        _tc_post_body,
        grid=(NA // BM,),
        in_specs=[_row_spec(32), _row_spec(32), _row_spec(1), _row_spec(1),
                  _row_spec(32), _full_spec(32, 16), _full_spec(1, 16),
                  _full_spec(16, 2), _full_spec(1, 2)],
        out_specs=_row_spec(2),
        out_shape=jax.ShapeDtypeStruct((NA, 2), F32),
    )(a0, a1, d0, d1, s2, w1, b1, w2, b2)


def kernel(x, edge_index, W1l, W1r, b1, W2l, W2r, b2, Wc1, bc1, Wc2, bc2):
    x = x.astype(F32)
    ei = edge_index.astype(I32)
    src = jnp.concatenate([ei[0], jnp.zeros((EP - E,), I32)]).reshape(NW * K, C)
    dst = jnp.concatenate([ei[1], jnp.full((EP - E,), JUNK, I32)]).reshape(NW * K, C)
    xp = jnp.pad(x, ((0, NA - N), (0, 0)))

    p1, s1 = _tc_pre(xp, W1l.T, W1r.T, b1.reshape(1, -1))
    agg1, deg = _sc_segment_sum(p1, src, dst, 64, with_deg=True)
    d0 = deg[0].reshape(NA, 1)
    d1 = deg[1].reshape(NA, 1)
    p2, s2 = _tc_mid(agg1[0], agg1[1], d0, d1, s1,
                     W2l.T, W2r.T, b2.reshape(1, -1))
    (agg2,) = _sc_segment_sum(p2, src, dst, 32, with_deg=False)
    out = _tc_post(agg2[0], agg2[1], d0, d1, s2,
                   Wc1.T, bc1.reshape(1, -1), Wc2.T, bc2.reshape(1, -1))
    return out[:N]


# speedup vs baseline: 13.2149x; 1.9748x over previous
import functools

import jax
import jax.numpy as jnp
from jax import lax
from jax.experimental import pallas as pl
from jax.experimental.pallas import tpu as pltpu
from jax.experimental.pallas import tpu_sc as plsc

F32 = jnp.float32
I32 = jnp.int32

N = 10000
NA = 10240
E = 320000
C = 128
NCORE = 2
NSUB = 16
NW = NCORE * NSUB
K = 80
EP = NW * C * K
JUNK = NA - 8
RPS = NA // NSUB
BM = 1024


def _sc_body(with_deg, d, *refs):
    if with_deg:
        (p_hbm, src_hbm, dst_hbm, zrow_hbm, zvec_hbm,
         agg_hbm, deg_hbm,
         accum_s, p_s, src_v, dst_v, rows0, rows1,
         gsem0, gsem1, ssem0, ssem1, deg_s, ones_v) = refs
    else:
        (p_hbm, src_hbm, dst_hbm, zrow_hbm,
         agg_hbm,
         accum_s, p_s, src_v, dst_v, rows0, rows1,
         gsem0, gsem1, ssem0, ssem1) = refs
    rows = (rows0, rows1)
    gsem = (gsem0, gsem1)
    ssem = (ssem0, ssem1)
    c = lax.axis_index("c")
    s = lax.axis_index("s")
    wid = c * NSUB + s

    zbase = s * RPS
    for j in range(RPS // 128):
        pltpu.sync_copy(zrow_hbm, accum_s.at[pl.ds(zbase + j * 128, 128)])
    if with_deg:
        pltpu.sync_copy(zvec_hbm, deg_s.at[pl.ds(zbase, RPS)])
        for j in range(C // 16):
            ones_v[pl.ds(j * 16, 16)] = jnp.full((16,), 1.0, F32)

    pltpu.sync_copy(p_hbm.at[pl.ds(s * RPS, RPS)], p_s.at[pl.ds(s * RPS, RPS)])
    pltpu.sync_copy(src_hbm.at[pl.ds(wid * K, K)], src_v)
    pltpu.sync_copy(dst_hbm.at[pl.ds(wid * K, K)], dst_v)
    plsc.subcore_barrier()

    pltpu.async_copy(p_s.at[src_v.at[0]], rows0, gsem0)

    def outer(j2, carry):
        for b in range(2):
            j = j2 * 2 + b
            pltpu.make_async_copy(p_s.at[src_v.at[j]], rows[b],
                                  gsem[b]).wait()
            pltpu.async_copy(rows[b], accum_s.at[dst_v.at[j]], ssem[b],
                             add=True)
            if with_deg:
                pltpu.sync_copy(ones_v, deg_s.at[dst_v.at[j]], add=True)

            @pl.when(j + 1 < K)
            def _issue():
                @pl.when(j >= 1)
                def _drain():
                    pltpu.make_async_copy(rows[1 - b],
                                          accum_s.at[dst_v.at[j - 1]],
                                          ssem[1 - b]).wait()
                pltpu.async_copy(p_s.at[src_v.at[j + 1]], rows[1 - b],
                                 gsem[1 - b])
        return carry

    lax.fori_loop(0, K // 2, outer, 0)
    pltpu.make_async_copy(rows0, accum_s.at[dst_v.at[K - 2]], ssem0).wait()
    pltpu.make_async_copy(rows1, accum_s.at[dst_v.at[K - 1]], ssem1).wait()
    plsc.subcore_barrier()

    wb = s * RPS
    pltpu.sync_copy(accum_s.at[pl.ds(wb, RPS)], agg_hbm.at[c, pl.ds(wb, RPS)])
    if with_deg:
        pltpu.sync_copy(deg_s.at[pl.ds(wb, RPS)], deg_hbm.at[c, pl.ds(wb, RPS)])


def _sc_segment_sum(p, src, dst, d, with_deg):
    mesh = plsc.VectorSubcoreMesh(core_axis_name="c", subcore_axis_name="s",
                                  num_cores=NCORE, num_subcores=NSUB)
    outs = [jax.ShapeDtypeStruct((NCORE, NA, d), F32)]
    scratch = [
        pltpu.VMEM_SHARED((NA, d), F32),
        pltpu.VMEM_SHARED((NA, d), F32),
        pltpu.VMEM((K, C), I32),
        pltpu.VMEM((K, C), I32),
        pltpu.VMEM((C, d), F32),
        pltpu.VMEM((C, d), F32),
        pltpu.SemaphoreType.DMA,
        pltpu.SemaphoreType.DMA,
        pltpu.SemaphoreType.DMA,
        pltpu.SemaphoreType.DMA,
    ]
    operands = [p, src, dst, jnp.zeros((128, d), F32)]
    if with_deg:
        outs.append(jax.ShapeDtypeStruct((NCORE, NA), F32))
        scratch += [pltpu.VMEM_SHARED((NA,), F32),
                    pltpu.VMEM((C,), F32)]
        operands.append(jnp.zeros((RPS,), F32))
    fn = pl.kernel(functools.partial(_sc_body, with_deg, d),
                   out_type=tuple(outs), mesh=mesh, scratch_types=scratch,
                   compiler_params=pltpu.CompilerParams(
                       use_tc_tiling_on_sc=False))
    return fn(*operands)


def _tc_pre_body(x_ref, wl_ref, wr_ref, b_ref, p_ref, s_ref):
    xb = x_ref[...]
    p_ref[...] = jnp.dot(xb, wl_ref[...], preferred_element_type=F32)
    s_ref[...] = jnp.dot(xb, wr_ref[...], preferred_element_type=F32) + b_ref[...]


def _tc_mid_body(a0, a1, d0, d1, s1, wl, wr, b, p_ref, s_ref):
    inv = 1.0 / jnp.maximum(d0[...] + d1[...], 1.0)
    h = jnp.maximum((a0[...] + a1[...]) * inv + s1[...], 0.0)
    p_ref[...] = jnp.dot(h, wl[...], preferred_element_type=F32)
    s_ref[...] = jnp.dot(h, wr[...], preferred_element_type=F32) + b[...]


def _tc_post_body(a0, a1, d0, d1, s2, w1, b1, w2, b2, o_ref):
    inv = 1.0 / jnp.maximum(d0[...] + d1[...], 1.0)
    h = jnp.maximum((a0[...] + a1[...]) * inv + s2[...], 0.0)
    h = jnp.maximum(jnp.dot(h, w1[...], preferred_element_type=F32) + b1[...], 0.0)
    o_ref[...] = jnp.dot(h, w2[...], preferred_element_type=F32) + b2[...]


def _row_spec(dcols):
    return pl.BlockSpec((BM, dcols), lambda i: (i, 0))


def _full_spec(r, c):
    return pl.BlockSpec((r, c), lambda i: (0, 0))


def _tc_pre(xp, wl, wr, b):
    return pl.pallas_call(
        _tc_pre_body,
        grid=(NA // BM,),
        in_specs=[_row_spec(128), _full_spec(128, 64), _full_spec(128, 64),
                  _full_spec(1, 64)],
        out_specs=[_row_spec(64), _row_spec(64)],
        out_shape=[jax.ShapeDtypeStruct((NA, 64), F32)] * 2,
    )(xp, wl, wr, b)


def _tc_mid(a0, a1, d0, d1, s1, wl, wr, b):
    return pl.pallas_call(
        _tc_mid_body,
        grid=(NA // BM,),
        in_specs=[_row_spec(64), _row_spec(64), _row_spec(1), _row_spec(1),
                  _row_spec(64), _full_spec(64, 32), _full_spec(64, 32),
                  _full_spec(1, 32)],
        out_specs=[_row_spec(32), _row_spec(32)],
        out_shape=[jax.ShapeDtypeStruct((NA, 32), F32)] * 2,
    )(a0, a1, d0, d1, s1, wl, wr, b)


def _tc_post(a0, a1, d0, d1, s2, w1, b1, w2, b2):
    return pl.pallas_call(
        _tc_post_body,
        grid=(NA // BM,),
        in_specs=[_row_spec(32), _row_spec(32), _row_spec(1), _row_spec(1),
                  _row_spec(32), _full_spec(32, 16), _full_spec(1, 16),
                  _full_spec(16, 2), _full_spec(1, 2)],
        out_specs=_row_spec(2),
        out_shape=jax.ShapeDtypeStruct((NA, 2), F32),
    )(a0, a1, d0, d1, s2, w1, b1, w2, b2)


def kernel(x, edge_index, W1l, W1r, b1, W2l, W2r, b2, Wc1, bc1, Wc2, bc2):
    x = x.astype(F32)
    ei = edge_index.astype(I32)
    src = jnp.concatenate([ei[0], jnp.zeros((EP - E,), I32)]).reshape(NW * K, C)
    dst = jnp.concatenate([ei[1], jnp.full((EP - E,), JUNK, I32)]).reshape(NW * K, C)
    xp = jnp.pad(x, ((0, NA - N), (0, 0)))

    p1, s1 = _tc_pre(xp, W1l.T, W1r.T, b1.reshape(1, -1))
    agg1, deg = _sc_segment_sum(p1, src, dst, 64, with_deg=True)
    d0 = deg[0].reshape(NA, 1)
    d1 = deg[1].reshape(NA, 1)
    p2, s2 = _tc_mid(agg1[0], agg1[1], d0, d1, s1,
                     W2l.T, W2r.T, b2.reshape(1, -1))
    (agg2,) = _sc_segment_sum(p2, src, dst, 32, with_deg=False)
    out = _tc_post(agg2[0], agg2[1], d0, d1, s2,
                   Wc1.T, bc1.reshape(1, -1), Wc2.T, bc2.reshape(1, -1))
    return out[:N]


# speedup vs baseline: 13.4626x; 1.0187x over previous
import functools

import jax
import jax.numpy as jnp
from jax import lax
from jax.experimental import pallas as pl
from jax.experimental.pallas import tpu as pltpu
from jax.experimental.pallas import tpu_sc as plsc

F32 = jnp.float32
I32 = jnp.int32

N = 10000
NA = 10240
E = 320000
C = 128
NCORE = 2
NSUB = 16
NW = NCORE * NSUB
K = 80
EP = NW * C * K
JUNK = NA - 8
RPS = NA // NSUB
PRS = N // NSUB
BM = 1000


def _sc_body(with_deg, d, *refs):
    if with_deg:
        (p_hbm, src_hbm, dst_hbm, zrow_hbm, zvec_hbm,
         agg_hbm, deg_hbm,
         accum_s, p_s, src_v, dst_v, rows0, rows1,
         gsem0, gsem1, ssem0, ssem1, deg_s, ones_v) = refs
    else:
        (p_hbm, src_hbm, dst_hbm, zrow_hbm,
         agg_hbm,
         accum_s, p_s, src_v, dst_v, rows0, rows1,
         gsem0, gsem1, ssem0, ssem1) = refs
    rows = (rows0, rows1)
    gsem = (gsem0, gsem1)
    ssem = (ssem0, ssem1)
    c = lax.axis_index("c")
    s = lax.axis_index("s")
    wid = c * NSUB + s

    zbase = s * RPS
    for j in range(RPS // 128):
        pltpu.sync_copy(zrow_hbm, accum_s.at[pl.ds(zbase + j * 128, 128)])
    if with_deg:
        pltpu.sync_copy(zvec_hbm, deg_s.at[pl.ds(zbase, RPS)])
        for j in range(C // 16):
            ones_v[pl.ds(j * 16, 16)] = jnp.full((16,), 1.0, F32)

    pltpu.sync_copy(p_hbm.at[pl.ds(s * PRS, PRS)], p_s.at[pl.ds(s * PRS, PRS)])
    pltpu.sync_copy(src_hbm.at[pl.ds(wid * K, K)], src_v)
    pltpu.sync_copy(dst_hbm.at[pl.ds(wid * K, K)], dst_v)
    plsc.subcore_barrier()

    pltpu.async_copy(p_s.at[src_v.at[0]], rows0, gsem0)

    def outer(j2, carry):
        for b in range(2):
            j = j2 * 2 + b
            pltpu.make_async_copy(p_s.at[src_v.at[j]], rows[b],
                                  gsem[b]).wait()
            pltpu.async_copy(rows[b], accum_s.at[dst_v.at[j]], ssem[b],
                             add=True)
            if with_deg:
                pltpu.sync_copy(ones_v, deg_s.at[dst_v.at[j]], add=True)

            @pl.when(j + 1 < K)
            def _issue():
                @pl.when(j >= 1)
                def _drain():
                    pltpu.make_async_copy(rows[1 - b],
                                          accum_s.at[dst_v.at[j - 1]],
                                          ssem[1 - b]).wait()
                pltpu.async_copy(p_s.at[src_v.at[j + 1]], rows[1 - b],
                                 gsem[1 - b])
        return carry

    lax.fori_loop(0, K // 2, outer, 0)
    pltpu.make_async_copy(rows0, accum_s.at[dst_v.at[K - 2]], ssem0).wait()
    pltpu.make_async_copy(rows1, accum_s.at[dst_v.at[K - 1]], ssem1).wait()
    plsc.subcore_barrier()

    wb = s * RPS
    pltpu.sync_copy(accum_s.at[pl.ds(wb, RPS)], agg_hbm.at[c, pl.ds(wb, RPS)])
    if with_deg:
        pltpu.sync_copy(deg_s.at[pl.ds(wb, RPS)], deg_hbm.at[c, pl.ds(wb, RPS)])


def _sc_segment_sum(p, src, dst, d, with_deg):
    mesh = plsc.VectorSubcoreMesh(core_axis_name="c", subcore_axis_name="s",
                                  num_cores=NCORE, num_subcores=NSUB)
    outs = [jax.ShapeDtypeStruct((NCORE, NA, d), F32)]
    scratch = [
        pltpu.VMEM_SHARED((NA, d), F32),
        pltpu.VMEM_SHARED((NA, d), F32),
        pltpu.VMEM((K, C), I32),
        pltpu.VMEM((K, C), I32),
        pltpu.VMEM((C, d), F32),
        pltpu.VMEM((C, d), F32),
        pltpu.SemaphoreType.DMA,
        pltpu.SemaphoreType.DMA,
        pltpu.SemaphoreType.DMA,
        pltpu.SemaphoreType.DMA,
    ]
    operands = [p, src, dst, jnp.zeros((128, d), F32)]
    if with_deg:
        outs.append(jax.ShapeDtypeStruct((NCORE, NA), F32))
        scratch += [pltpu.VMEM_SHARED((NA,), F32),
                    pltpu.VMEM((C,), F32)]
        operands.append(jnp.zeros((RPS,), F32))
    fn = pl.kernel(functools.partial(_sc_body, with_deg, d),
                   out_type=tuple(outs), mesh=mesh, scratch_types=scratch,
                   compiler_params=pltpu.CompilerParams(
                       use_tc_tiling_on_sc=False))
    return fn(*operands)


def _dotT(a, w):
    return lax.dot_general(a, w, (((1,), (1,)), ((), ())),
                           preferred_element_type=F32)


def _tc_pre_body(x_ref, wl_ref, wr_ref, b_ref, p_ref, s_ref):
    xb = x_ref[...]
    p_ref[...] = _dotT(xb, wl_ref[...])
    s_ref[...] = _dotT(xb, wr_ref[...]) + b_ref[...][None, :]


def _tc_mid_body(a0, a1, d0, d1, s1, wl, wr, b, p_ref, s_ref):
    inv = 1.0 / jnp.maximum(d0[...] + d1[...], 1.0)
    h = jnp.maximum((a0[...] + a1[...]) * inv + s1[...], 0.0)
    p_ref[...] = _dotT(h, wl[...])
    s_ref[...] = _dotT(h, wr[...]) + b[...][None, :]


def _tc_post_body(a0, a1, d0, d1, s2, w1, b1, w2, b2, o_ref):
    inv = 1.0 / jnp.maximum(d0[...] + d1[...], 1.0)
    h = jnp.maximum((a0[...] + a1[...]) * inv + s2[...], 0.0)
    h = jnp.maximum(_dotT(h, w1[...]) + b1[...][None, :], 0.0)
    o_ref[...] = _dotT(h, w2[...]) + b2[...][None, :]


def _row_spec(dcols):
    return pl.BlockSpec((BM, dcols), lambda i: (i, 0))


def _full_spec(r, c):
    return pl.BlockSpec((r, c), lambda i: (0, 0))


def _vec_spec(n):
    return pl.BlockSpec((n,), lambda i: (0,))


def _tc_pre(xp, wl, wr, b):
    return pl.pallas_call(
        _tc_pre_body,
        grid=(NA // BM,),
        in_specs=[_row_spec(128), _full_spec(64, 128), _full_spec(64, 128),
                  _vec_spec(64)],
        out_specs=[_row_spec(64), _row_spec(64)],
        out_shape=[jax.ShapeDtypeStruct((N, 64), F32)] * 2,
    )(xp, wl, wr, b)


def _tc_mid(a0, a1, d0, d1, s1, wl, wr, b):
    return pl.pallas_call(
        _tc_mid_body,
        grid=(NA // BM,),
        in_specs=[_row_spec(64), _row_spec(64), _row_spec(1), _row_spec(1),
                  _row_spec(64), _full_spec(32, 64), _full_spec(32, 64),
                  _vec_spec(32)],
        out_specs=[_row_spec(32), _row_spec(32)],
        out_shape=[jax.ShapeDtypeStruct((N, 32), F32)] * 2,
    )(a0, a1, d0, d1, s1, wl, wr, b)


def _tc_post(a0, a1, d0, d1, s2, w1, b1, w2, b2):
    return pl.pallas_call(
        _tc_post_body,
        grid=(NA // BM,),
        in_specs=[_row_spec(32), _row_spec(32), _row_spec(1), _row_spec(1),
                  _row_spec(32), _full_spec(16, 32), _vec_spec(16),
                  _full_spec(2, 16), _vec_spec(2)],
        out_specs=_row_spec(2),
        out_shape=jax.ShapeDtypeStruct((N, 2), F32),
    )(a0, a1, d0, d1, s2, w1, b1, w2, b2)


def kernel(x, edge_index, W1l, W1r, b1, W2l, W2r, b2, Wc1, bc1, Wc2, bc2):
    x = x.astype(F32)
    ei = edge_index.astype(I32)
    src = jnp.concatenate([ei[0], jnp.zeros((EP - E,), I32)]).reshape(NW * K, C)
    dst = jnp.concatenate([ei[1], jnp.full((EP - E,), JUNK, I32)]).reshape(NW * K, C)
    p1, s1 = _tc_pre(x, W1l, W1r, b1)
    agg1, deg = _sc_segment_sum(p1, src, dst, 64, with_deg=True)
    d0 = deg[0].reshape(NA, 1)
    d1 = deg[1].reshape(NA, 1)
    p2, s2 = _tc_mid(agg1[0], agg1[1], d0, d1, s1, W2l, W2r, b2)
    (agg2,) = _sc_segment_sum(p2, src, dst, 32, with_deg=False)
    return _tc_post(agg2[0], agg2[1], d0, d1, s2, Wc1, bc1, Wc2, bc2)
